# Initial kernel scaffold; baseline (speedup 1.0000x reference)
import jax
import jax.numpy as jnp
from jax.experimental import pallas as pl


def kernel(x, edge_index, enc_W, enc_b, t, W1, b1, g1, be1, W2, b2, ln_g, ln_b, lin_W, lin_b):
    raise NotImplementedError("write your pallas kernel here")


# speedup vs baseline: 5.6112x; 5.6112x over previous
import functools

import jax
import jax.numpy as jnp
from jax import lax
from jax.experimental import pallas as pl
from jax.experimental.pallas import tpu as pltpu
from jax.experimental.pallas import tpu_sc as plsc

N = 10000
E = 320000
H = 128
L = 14
NCLS = 40
EPS = 1e-7

NSUB = 16
CHUNK = 128
GSZ = 16
GPT = 10
CPT = GPT * GSZ
E_PAD = NSUB * CPT * CHUNK
ROWS_PT = 632
ACC_ROWS = NSUB * ROWS_PT

BN = 1000
GRID = N // BN

def _sc_edge_body(ea_hbm, mea_hbm, src_hbm, dst_hbm, zrows_hbm, out_hbm,
                  src_v, dst_v, buf, acc, sem):
    cid = lax.axis_index("c")
    sid = lax.axis_index("s")
    pltpu.sync_copy(zrows_hbm, acc.at[pl.ds(sid * ROWS_PT, ROWS_PT)])
    plsc.subcore_barrier()

    def run(tbl):
        def group(g, carry):
            pltpu.sync_copy(src_hbm.at[sid, g], src_v)
            pltpu.sync_copy(dst_hbm.at[sid, g], dst_v)

            def body(k, c2):
                pltpu.async_copy(tbl.at[src_v.at[k]], buf, sem).wait()
                pltpu.sync_copy(buf, acc.at[dst_v.at[k]], add=True)
                return c2
            return lax.fori_loop(0, GSZ, body, carry)
        lax.fori_loop(0, GPT, group, 0)

    @pl.when(cid == 0)
    def _():
        run(ea_hbm)

    @pl.when(cid == 1)
    def _():
        run(mea_hbm)

    plsc.subcore_barrier()
    pltpu.sync_copy(acc.at[pl.ds(sid * ROWS_PT, ROWS_PT)],
                    out_hbm.at[cid, pl.ds(sid * ROWS_PT, ROWS_PT)])


@functools.cache
def _sc_edge():
    mesh = plsc.VectorSubcoreMesh(core_axis_name="c", subcore_axis_name="s",
                                  num_cores=2, num_subcores=NSUB)
    return pl.kernel(
        _sc_edge_body,
        out_type=jax.ShapeDtypeStruct((2, ACC_ROWS, H), jnp.float32),
        mesh=mesh,
        scratch_types=[
            pltpu.VMEM((GSZ, CHUNK), jnp.int32),
            pltpu.VMEM((GSZ, CHUNK), jnp.int32),
            pltpu.VMEM((CHUNK, H), jnp.float32),
            pltpu.VMEM_SHARED((ACC_ROWS, H), jnp.float32),
            pltpu.SemaphoreType.DMA,
        ],
    )


def _dot(a, b):
    return lax.dot_general(a, b, (((1,), (0,)), ((), ())),
                           precision=lax.Precision.HIGHEST,
                           preferred_element_type=jnp.float32)


def _ln_rows(v, g, b):
    mu = jnp.mean(v, axis=-1, keepdims=True)
    var = jnp.mean((v - mu) ** 2, axis=-1, keepdims=True)
    return (v - mu) * lax.rsqrt(var + 1e-5) * g + b


def _tables(r, trow, ea_ref, mea_ref):
    a = jnp.maximum(r, 0.0) + EPS
    ea = jnp.exp(a * trow)
    ea_ref[...] = ea
    mea_ref[...] = a * ea


def _enc_body(x_ref, w_ref, b_ref, t_ref, h_ref, ea_ref, mea_ref):
    h = _dot(x_ref[...], w_ref[...]) + b_ref[...]
    h_ref[...] = h
    _tables(h, t_ref[...], ea_ref, mea_ref)


def _mlp(acc_ref, xin_ref, hprev_ref, w1_ref, b1_ref, g1_ref, be1_ref,
         w2_ref, b2_ref):
    den = acc_ref[0]
    num = acc_ref[1]
    out = num / (den + 1e-16) + xin_ref[...]
    hh = _dot(out, w1_ref[...]) + b1_ref[...]
    hh = jnp.maximum(_ln_rows(hh, g1_ref[...], be1_ref[...]), 0.0)
    return hprev_ref[...] + _dot(hh, w2_ref[...]) + b2_ref[...]


def _mid_body(acc_ref, xin_ref, hprev_ref, w1_ref, b1_ref, g1_ref, be1_ref,
              w2_ref, b2_ref, lng_ref, lnb_ref, t_ref,
              h_ref, r_ref, ea_ref, mea_ref):
    h = _mlp(acc_ref, xin_ref, hprev_ref, w1_ref, b1_ref, g1_ref, be1_ref,
             w2_ref, b2_ref)
    h_ref[...] = h
    r = jnp.maximum(_ln_rows(h, lng_ref[...], lnb_ref[...]), 0.0)
    r_ref[...] = r
    _tables(r, t_ref[...], ea_ref, mea_ref)


def _fin_body(acc_ref, xin_ref, hprev_ref, w1_ref, b1_ref, g1_ref, be1_ref,
              w2_ref, b2_ref, lng_ref, lnb_ref, lw_ref, lb_ref, o_ref):
    h = _mlp(acc_ref, xin_ref, hprev_ref, w1_ref, b1_ref, g1_ref, be1_ref,
             w2_ref, b2_ref)
    f = jnp.maximum(_ln_rows(h, lng_ref[...], lnb_ref[...]), 0.0)
    logits = _dot(f, lw_ref[...]) + lb_ref[...]
    m = jnp.max(logits, axis=-1, keepdims=True)
    s = logits - m
    o_ref[...] = s - jnp.log(jnp.sum(jnp.exp(s), axis=-1, keepdims=True))


def _row_spec(cols):
    return pl.BlockSpec((BN, cols), lambda i: (i, 0))


def _full_spec(shape):
    nd = len(shape)
    return pl.BlockSpec(shape, lambda i, _n=nd: (0,) * _n)


_ACC_SPEC = pl.BlockSpec((2, BN, H), lambda i: (0, i, 0))

_enc_call = pl.pallas_call(
    _enc_body,
    grid=(GRID,),
    in_specs=[_row_spec(H), _full_spec((H, H)), _full_spec((1, H)),
              _full_spec((1, H))],
    out_specs=[_row_spec(H)] * 3,
    out_shape=[jax.ShapeDtypeStruct((N, H), jnp.float32)] * 3,
)

_mid_call = pl.pallas_call(
    _mid_body,
    grid=(GRID,),
    in_specs=[_ACC_SPEC, _row_spec(H), _row_spec(H),
              _full_spec((H, 2 * H)), _full_spec((1, 2 * H)),
              _full_spec((1, 2 * H)), _full_spec((1, 2 * H)),
              _full_spec((2 * H, H)), _full_spec((1, H)),
              _full_spec((1, H)), _full_spec((1, H)), _full_spec((1, H))],
    out_specs=[_row_spec(H)] * 4,
    out_shape=[jax.ShapeDtypeStruct((N, H), jnp.float32)] * 4,
)

_fin_call = pl.pallas_call(
    _fin_body,
    grid=(GRID,),
    in_specs=[_ACC_SPEC, _row_spec(H), _row_spec(H),
              _full_spec((H, 2 * H)), _full_spec((1, 2 * H)),
              _full_spec((1, 2 * H)), _full_spec((1, 2 * H)),
              _full_spec((2 * H, H)), _full_spec((1, H)),
              _full_spec((1, H)), _full_spec((1, H)),
              _full_spec((H, H)), _full_spec((1, H))],
    out_specs=[_row_spec(H)],
    out_shape=[jax.ShapeDtypeStruct((N, H), jnp.float32)],
)


def kernel(x, edge_index, enc_W, enc_b, t, W1, b1, g1, be1, W2, b2,
           ln_g, ln_b, lin_W, lin_b):
    f32 = jnp.float32
    src = edge_index[0].astype(jnp.int32)
    dst = edge_index[1].astype(jnp.int32)
    pad = E_PAD - E
    src_p = jnp.concatenate([src, jnp.zeros((pad,), jnp.int32)]
                            ).reshape(NSUB, GPT, GSZ, CHUNK)
    dst_p = jnp.concatenate([dst, jnp.full((pad,), N, jnp.int32)]
                            ).reshape(NSUB, GPT, GSZ, CHUNK)
    zrows = jnp.zeros((ROWS_PT, H), f32)
    ones_row = jnp.ones((1, H), f32)
    trows = t.reshape(L, 1) * ones_row

    lw_pad = jnp.pad(lin_W, ((0, 0), (0, H - NCLS)))
    lb_pad = jnp.concatenate([lin_b, jnp.full((H - NCLS,), -1e30, f32)]
                             ).reshape(1, H)

    h0, ea, mea = _enc_call(x, enc_W, enc_b.reshape(1, H), trows[0:1])
    xin = h0
    hprev = jnp.zeros((N, H), f32)
    sc_edge = _sc_edge()
    for l in range(L):
        acc = sc_edge(ea, mea, src_p, dst_p, zrows)
        r1 = lambda a: a.reshape(1, -1)
        if l < L - 1:
            hprev, xin, ea, mea = _mid_call(
                acc, xin, hprev, W1[l], r1(b1[l]), r1(g1[l]), r1(be1[l]),
                W2[l], r1(b2[l]), r1(ln_g[l + 1]), r1(ln_b[l + 1]),
                trows[l + 1:l + 2])
        else:
            (opad,) = _fin_call(
                acc, xin, hprev, W1[l], r1(b1[l]), r1(g1[l]), r1(be1[l]),
                W2[l], r1(b2[l]), r1(ln_g[0]), r1(ln_b[0]), lw_pad, lb_pad)
    return opad[:, :NCLS]


# speedup vs baseline: 6.2705x; 1.1175x over previous
import functools

import jax
import jax.numpy as jnp
from jax import lax
from jax.experimental import pallas as pl
from jax.experimental.pallas import tpu as pltpu
from jax.experimental.pallas import tpu_sc as plsc

N = 10000
E = 320000
H = 128
L = 14
NCLS = 40
EPS = 1e-7

NSUB = 16
CHUNK = 128
GSZ = 16
GPT = 10
CPT = GPT * GSZ
E_PAD = NSUB * CPT * CHUNK
ROWS_PT = 632
ACC_ROWS = NSUB * ROWS_PT

BN = 1000
GRID = N // BN

def _sc_edge_body(ea_hbm, mea_hbm, src_hbm, dst_hbm, zrows_hbm, out_hbm,
                  src_v, dst_v, buf0, buf1, acc, gsem0, gsem1, ssem0, ssem1):
    cid = lax.axis_index("c")
    sid = lax.axis_index("s")
    bufs = (buf0, buf1)
    gsems = (gsem0, gsem1)
    ssems = (ssem0, ssem1)
    pltpu.sync_copy(zrows_hbm, acc.at[pl.ds(sid * ROWS_PT, ROWS_PT)])
    plsc.subcore_barrier()

    def run(tbl):
        def group(g, carry):
            pltpu.sync_copy(src_hbm.at[sid, g], src_v)
            pltpu.sync_copy(dst_hbm.at[sid, g], dst_v)

            def pair(k2, c2):
                for i in range(2):
                    k = k2 * 2 + i

                    @pl.when(k2 > 0)
                    def _():
                        pltpu.make_async_copy(
                            bufs[i], acc.at[dst_v.at[k - 2]], ssems[i]
                        ).wait()
                    pltpu.async_copy(tbl.at[src_v.at[k]], bufs[i],
                                     gsems[i]).wait()
                    pltpu.async_copy(bufs[i], acc.at[dst_v.at[k]], ssems[i],
                                     add=True)
                return c2
            lax.fori_loop(0, GSZ // 2, pair, carry)
            for i in range(2):
                pltpu.make_async_copy(bufs[i], acc.at[dst_v.at[GSZ - 2 + i]],
                                      ssems[i]).wait()
            return carry
        lax.fori_loop(0, GPT, group, 0)

    @pl.when(cid == 0)
    def _():
        run(ea_hbm)

    @pl.when(cid == 1)
    def _():
        run(mea_hbm)

    plsc.subcore_barrier()
    pltpu.sync_copy(acc.at[pl.ds(sid * ROWS_PT, ROWS_PT)],
                    out_hbm.at[cid, pl.ds(sid * ROWS_PT, ROWS_PT)])


@functools.cache
def _sc_edge():
    mesh = plsc.VectorSubcoreMesh(core_axis_name="c", subcore_axis_name="s",
                                  num_cores=2, num_subcores=NSUB)
    return pl.kernel(
        _sc_edge_body,
        out_type=jax.ShapeDtypeStruct((2, ACC_ROWS, H), jnp.float32),
        mesh=mesh,
        scratch_types=[
            pltpu.VMEM((GSZ, CHUNK), jnp.int32),
            pltpu.VMEM((GSZ, CHUNK), jnp.int32),
            pltpu.VMEM((CHUNK, H), jnp.float32),
            pltpu.VMEM((CHUNK, H), jnp.float32),
            pltpu.VMEM_SHARED((ACC_ROWS, H), jnp.float32),
            pltpu.SemaphoreType.DMA,
            pltpu.SemaphoreType.DMA,
            pltpu.SemaphoreType.DMA,
            pltpu.SemaphoreType.DMA,
        ],
    )


def _dot(a, b):
    return lax.dot_general(a, b, (((1,), (0,)), ((), ())),
                           precision=lax.Precision.HIGHEST,
                           preferred_element_type=jnp.float32)


def _ln_rows(v, g, b):
    mu = jnp.mean(v, axis=-1, keepdims=True)
    var = jnp.mean((v - mu) ** 2, axis=-1, keepdims=True)
    return (v - mu) * lax.rsqrt(var + 1e-5) * g + b


def _tables(r, trow, ea_ref, mea_ref):
    a = jnp.maximum(r, 0.0) + EPS
    ea = jnp.exp(a * trow)
    ea_ref[...] = ea
    mea_ref[...] = a * ea


def _enc_body(x_ref, w_ref, b_ref, t_ref, h_ref, ea_ref, mea_ref):
    h = _dot(x_ref[...], w_ref[...]) + b_ref[...]
    h_ref[...] = h
    _tables(h, t_ref[...], ea_ref, mea_ref)


def _mlp(acc_ref, xin_ref, hprev_ref, w1_ref, b1_ref, g1_ref, be1_ref,
         w2_ref, b2_ref):
    den = acc_ref[0]
    num = acc_ref[1]
    out = num / (den + 1e-16) + xin_ref[...]
    hh = _dot(out, w1_ref[...]) + b1_ref[...]
    hh = jnp.maximum(_ln_rows(hh, g1_ref[...], be1_ref[...]), 0.0)
    return hprev_ref[...] + _dot(hh, w2_ref[...]) + b2_ref[...]


def _mid_body(acc_ref, xin_ref, hprev_ref, w1_ref, b1_ref, g1_ref, be1_ref,
              w2_ref, b2_ref, lng_ref, lnb_ref, t_ref,
              h_ref, r_ref, ea_ref, mea_ref):
    h = _mlp(acc_ref, xin_ref, hprev_ref, w1_ref, b1_ref, g1_ref, be1_ref,
             w2_ref, b2_ref)
    h_ref[...] = h
    r = jnp.maximum(_ln_rows(h, lng_ref[...], lnb_ref[...]), 0.0)
    r_ref[...] = r
    _tables(r, t_ref[...], ea_ref, mea_ref)


def _fin_body(acc_ref, xin_ref, hprev_ref, w1_ref, b1_ref, g1_ref, be1_ref,
              w2_ref, b2_ref, lng_ref, lnb_ref, lw_ref, lb_ref, o_ref):
    h = _mlp(acc_ref, xin_ref, hprev_ref, w1_ref, b1_ref, g1_ref, be1_ref,
             w2_ref, b2_ref)
    f = jnp.maximum(_ln_rows(h, lng_ref[...], lnb_ref[...]), 0.0)
    logits = _dot(f, lw_ref[...]) + lb_ref[...]
    m = jnp.max(logits, axis=-1, keepdims=True)
    s = logits - m
    o_ref[...] = s - jnp.log(jnp.sum(jnp.exp(s), axis=-1, keepdims=True))


def _row_spec(cols):
    return pl.BlockSpec((BN, cols), lambda i: (i, 0))


def _full_spec(shape):
    nd = len(shape)
    return pl.BlockSpec(shape, lambda i, _n=nd: (0,) * _n)


_ACC_SPEC = pl.BlockSpec((2, BN, H), lambda i: (0, i, 0))

_enc_call = pl.pallas_call(
    _enc_body,
    grid=(GRID,),
    in_specs=[_row_spec(H), _full_spec((H, H)), _full_spec((1, H)),
              _full_spec((1, H))],
    out_specs=[_row_spec(H)] * 3,
    out_shape=[jax.ShapeDtypeStruct((N, H), jnp.float32)] * 3,
)

_mid_call = pl.pallas_call(
    _mid_body,
    grid=(GRID,),
    in_specs=[_ACC_SPEC, _row_spec(H), _row_spec(H),
              _full_spec((H, 2 * H)), _full_spec((1, 2 * H)),
              _full_spec((1, 2 * H)), _full_spec((1, 2 * H)),
              _full_spec((2 * H, H)), _full_spec((1, H)),
              _full_spec((1, H)), _full_spec((1, H)), _full_spec((1, H))],
    out_specs=[_row_spec(H)] * 4,
    out_shape=[jax.ShapeDtypeStruct((N, H), jnp.float32)] * 4,
)

_fin_call = pl.pallas_call(
    _fin_body,
    grid=(GRID,),
    in_specs=[_ACC_SPEC, _row_spec(H), _row_spec(H),
              _full_spec((H, 2 * H)), _full_spec((1, 2 * H)),
              _full_spec((1, 2 * H)), _full_spec((1, 2 * H)),
              _full_spec((2 * H, H)), _full_spec((1, H)),
              _full_spec((1, H)), _full_spec((1, H)),
              _full_spec((H, H)), _full_spec((1, H))],
    out_specs=[_row_spec(H)],
    out_shape=[jax.ShapeDtypeStruct((N, H), jnp.float32)],
)


def kernel(x, edge_index, enc_W, enc_b, t, W1, b1, g1, be1, W2, b2,
           ln_g, ln_b, lin_W, lin_b):
    f32 = jnp.float32
    src = edge_index[0].astype(jnp.int32)
    dst = edge_index[1].astype(jnp.int32)
    pad = E_PAD - E
    src_p = jnp.concatenate([src, jnp.zeros((pad,), jnp.int32)]
                            ).reshape(NSUB, GPT, GSZ, CHUNK)
    dst_p = jnp.concatenate([dst, jnp.full((pad,), N, jnp.int32)]
                            ).reshape(NSUB, GPT, GSZ, CHUNK)
    zrows = jnp.zeros((ROWS_PT, H), f32)
    ones_row = jnp.ones((1, H), f32)
    trows = t.reshape(L, 1) * ones_row

    lw_pad = jnp.pad(lin_W, ((0, 0), (0, H - NCLS)))
    lb_pad = jnp.concatenate([lin_b, jnp.full((H - NCLS,), -1e30, f32)]
                             ).reshape(1, H)

    h0, ea, mea = _enc_call(x, enc_W, enc_b.reshape(1, H), trows[0:1])
    xin = h0
    hprev = jnp.zeros((N, H), f32)
    sc_edge = _sc_edge()
    for l in range(L):
        acc = sc_edge(ea, mea, src_p, dst_p, zrows)
        r1 = lambda a: a.reshape(1, -1)
        if l < L - 1:
            hprev, xin, ea, mea = _mid_call(
                acc, xin, hprev, W1[l], r1(b1[l]), r1(g1[l]), r1(be1[l]),
                W2[l], r1(b2[l]), r1(ln_g[l + 1]), r1(ln_b[l + 1]),
                trows[l + 1:l + 2])
        else:
            (opad,) = _fin_call(
                acc, xin, hprev, W1[l], r1(b1[l]), r1(g1[l]), r1(be1[l]),
                W2[l], r1(b2[l]), r1(ln_g[0]), r1(ln_b[0]), lw_pad, lb_pad)
    return opad[:, :NCLS]
